# baseline (speedup 1.0000x reference)
import functools

import jax
import jax.numpy as jnp
from jax import lax
from jax.experimental import pallas as pl
from jax.experimental.pallas import tpu as pltpu
from jax.experimental.pallas import tpu_sc as plsc

UNITS = 128
VOCAB = 1000
ROWS = VOCAB + 1
DPAD = 1024
BATCH = 4096
SEQ = 20
TOK = BATCH * SEQ

NC, NS, L = 2, 16, 16
NW = NC * NS
TPW = TOK // NW
CH = 64
NCH = TPW // CH


def _gram_body(w_ref, wp_ref, g_ref):
    g_ref[...] = lax.dot_general(
        w_ref[...], wp_ref[...], (((1,), (1,)), ((), ())),
        preferred_element_type=jnp.float32,
    )


def _gram(w):
    wp = jnp.pad(w, ((0, DPAD - ROWS), (0, 0)))
    return pl.pallas_call(
        _gram_body,
        out_shape=jax.ShapeDtypeStruct((ROWS, DPAD), jnp.float32),
    )(w, wp)


def _gather_body(x_hbm, g_hbm, out_hbm, xbuf, idxbuf, rowsbuf, sem):
    wid = lax.axis_index("s") * NC + lax.axis_index("c")
    base = wid * TPW

    def chunk(g, carry):
        off = base + g * CH
        pltpu.sync_copy(x_hbm.at[pl.ds(off, CH)], xbuf)
        for j in range(CH // L):
            sl = pl.ds(j * L, L)
            idxbuf[sl] = (xbuf[sl] * float(VOCAB)).astype(jnp.int32)
        pltpu.async_copy(g_hbm.at[idxbuf], rowsbuf, sem).wait()
        pltpu.sync_copy(rowsbuf, out_hbm.at[pl.ds(off, CH)])
        return carry

    lax.fori_loop(0, NCH, chunk, 0)


_gather = functools.partial(
    pl.kernel,
    out_type=jax.ShapeDtypeStruct((TOK, DPAD), jnp.float32),
    mesh=plsc.VectorSubcoreMesh(core_axis_name="c", subcore_axis_name="s"),
    scratch_types=[
        pltpu.VMEM((CH,), jnp.float32),
        pltpu.VMEM((CH,), jnp.int32),
        pltpu.VMEM((CH, DPAD), jnp.float32),
        pltpu.SemaphoreType.DMA,
    ],
)(_gather_body)


def kernel(x, c, we_item):
    del c
    g = _gram(we_item)
    x_flat = x.reshape(TOK)
    out = _gather(x_flat, g)
    return out[:, :ROWS].reshape(BATCH, SEQ, ROWS)


# speedup vs baseline: 1.0467x; 1.0467x over previous
import functools

import jax
import jax.numpy as jnp
from jax import lax
from jax.experimental import pallas as pl
from jax.experimental.pallas import tpu as pltpu
from jax.experimental.pallas import tpu_sc as plsc

UNITS = 128
VOCAB = 1000
ROWS = VOCAB + 1
DPAD = 1024
BATCH = 4096
SEQ = 20
TOK = BATCH * SEQ

NC, NS, L = 2, 16, 16
NW = NC * NS
TPW = TOK // NW
BPW = BATCH // NW
NPAIR = BPW // 2
CH = 2 * SEQ
XP = 160
NWIN = ROWS // L
TAIL = ROWS - L


def _gram_body(w_ref, wp_ref, g_ref):
    g_ref[...] = lax.dot_general(
        w_ref[...], wp_ref[...], (((1,), (1,)), ((), ())),
        preferred_element_type=jnp.float32,
    )


def _gram(w):
    wp = jnp.pad(w, ((0, DPAD - ROWS), (0, 0)))
    return pl.pallas_call(
        _gram_body,
        out_shape=jax.ShapeDtypeStruct((ROWS, DPAD), jnp.float32),
    )(w, wp)


def _gather_body(x_hbm, g_hbm, out_hbm,
                 xtmp, idxbuf, rows0, rows1, outbuf, sem0, sem1):
    wid = lax.axis_index("s") * NC + lax.axis_index("c")
    base = wid * TPW

    def piece(p, c):
        pltpu.sync_copy(x_hbm.at[pl.ds(base + p * XP, XP)], xtmp)

        def conv(j, c2):
            sl = pl.ds(j * L, L)
            idxbuf[pl.ds(p * XP + j * L, L)] = (
                xtmp[sl] * float(VOCAB)).astype(jnp.int32)
            return c2

        lax.fori_loop(0, XP // L, conv, 0)
        return c

    lax.fori_loop(0, TPW // XP, piece, 0)

    rows = (rows0, rows1)
    sems = (sem0, sem1)

    def issue(g, buf_i):
        idxsl = idxbuf.at[pl.ds(g * CH, CH)]
        pltpu.async_copy(g_hbm.at[idxsl], rows[buf_i], sems[buf_i])

    def drain_compact_store(g, buf_i):
        idxsl = idxbuf.at[pl.ds(g * CH, CH)]
        pltpu.make_async_copy(g_hbm.at[idxsl], rows[buf_i], sems[buf_i]).wait()

        for half in range(2):
            def row_copy(r, c):
                src = rows[buf_i].at[half * SEQ + r]
                for j in range(NWIN):
                    sl = pl.ds(j * L, L)
                    outbuf[r, sl] = src[sl]
                tl = pl.ds(TAIL, L)
                outbuf[r, tl] = src[tl]
                return c

            lax.fori_loop(0, SEQ, row_copy, 0)
            pltpu.sync_copy(outbuf, out_hbm.at[wid * BPW + 2 * g + half])

    issue(0, 0)

    def pair(i, carry):
        g0 = i * 2
        issue(g0 + 1, 1)
        drain_compact_store(g0, 0)

        @pl.when(i + 1 < NPAIR // 2)
        def _():
            issue(g0 + 2, 0)

        drain_compact_store(g0 + 1, 1)
        return carry

    lax.fori_loop(0, NPAIR // 2, pair, 0)


_gather = functools.partial(
    pl.kernel,
    out_type=jax.ShapeDtypeStruct((BATCH, SEQ, ROWS), jnp.float32),
    mesh=plsc.VectorSubcoreMesh(core_axis_name="c", subcore_axis_name="s"),
    scratch_types=[
        pltpu.VMEM((XP,), jnp.float32),
        pltpu.VMEM((TPW,), jnp.int32),
        pltpu.VMEM((CH, DPAD), jnp.float32),
        pltpu.VMEM((CH, DPAD), jnp.float32),
        pltpu.VMEM((SEQ, ROWS), jnp.float32),
        pltpu.SemaphoreType.DMA,
        pltpu.SemaphoreType.DMA,
    ],
)(_gather_body)


def kernel(x, c, we_item):
    del c
    g = _gram(we_item)
    x_flat = x.reshape(TOK)
    return _gather(x_flat, g)


# speedup vs baseline: 1.2349x; 1.1798x over previous
import functools

import jax
import jax.numpy as jnp
from jax import lax
from jax.experimental import pallas as pl
from jax.experimental.pallas import tpu as pltpu
from jax.experimental.pallas import tpu_sc as plsc

UNITS = 128
VOCAB = 1000
ROWS = VOCAB + 1
DPAD = 1024
BATCH = 4096
SEQ = 20
TOK = BATCH * SEQ

NC, NS, L = 2, 16, 16
NW = NC * NS
TPW = TOK // NW
BPW = BATCH // NW
NPAIR = BPW // 2
CH = 2 * SEQ
XP = 160
NWIN = ROWS // L
TAIL = ROWS - L


def _gram_body(w_ref, wp_ref, g_ref):
    g_ref[...] = lax.dot_general(
        w_ref[...], wp_ref[...], (((1,), (1,)), ((), ())),
        preferred_element_type=jnp.float32,
    )


def _gram(w):
    wp = jnp.pad(w, ((0, DPAD - ROWS), (0, 0)))
    return pl.pallas_call(
        _gram_body,
        out_shape=jax.ShapeDtypeStruct((ROWS, DPAD), jnp.float32),
    )(w, wp)


def _gather_body(x_hbm, g_hbm, out_hbm,
                 xtmp, idxbuf, rows0, rows1, outbuf, sem0, sem1):
    wid = lax.axis_index("s") * NC + lax.axis_index("c")
    base = wid * TPW

    def piece(p, c):
        pltpu.sync_copy(x_hbm.at[pl.ds(base + p * XP, XP)], xtmp)

        def conv(j, c2):
            sl = pl.ds(j * L, L)
            idxbuf[pl.ds(p * XP + j * L, L)] = (
                xtmp[sl] * float(VOCAB)).astype(jnp.int32)
            return c2

        lax.fori_loop(0, XP // L, conv, 0)
        return c

    lax.fori_loop(0, TPW // XP, piece, 0)

    rows = (rows0, rows1)
    sems = (sem0, sem1)

    def issue(g, buf_i):
        idxsl = idxbuf.at[pl.ds(g * CH, CH)]
        pltpu.async_copy(g_hbm.at[idxsl], rows[buf_i], sems[buf_i])

    def drain_compact_store(g, buf_i):
        idxsl = idxbuf.at[pl.ds(g * CH, CH)]
        pltpu.make_async_copy(g_hbm.at[idxsl], rows[buf_i], sems[buf_i]).wait()

        for half in range(2):
            @plsc.parallel_loop(0, SEQ, 1, unroll=2)
            def _(r):
                src = rows[buf_i].at[half * SEQ + r]
                for j in range(NWIN):
                    sl = pl.ds(j * L, L)
                    outbuf[r, sl] = src[sl]
                tl = pl.ds(TAIL, L)
                outbuf[r, tl] = src[tl]

            pltpu.sync_copy(outbuf, out_hbm.at[wid * BPW + 2 * g + half])

    issue(0, 0)

    def pair(i, carry):
        g0 = i * 2
        issue(g0 + 1, 1)
        drain_compact_store(g0, 0)

        @pl.when(i + 1 < NPAIR // 2)
        def _():
            issue(g0 + 2, 0)

        drain_compact_store(g0 + 1, 1)
        return carry

    lax.fori_loop(0, NPAIR // 2, pair, 0)


_gather = functools.partial(
    pl.kernel,
    out_type=jax.ShapeDtypeStruct((BATCH, SEQ, ROWS), jnp.float32),
    mesh=plsc.VectorSubcoreMesh(core_axis_name="c", subcore_axis_name="s"),
    scratch_types=[
        pltpu.VMEM((XP,), jnp.float32),
        pltpu.VMEM((TPW,), jnp.int32),
        pltpu.VMEM((CH, DPAD), jnp.float32),
        pltpu.VMEM((CH, DPAD), jnp.float32),
        pltpu.VMEM((SEQ, ROWS), jnp.float32),
        pltpu.SemaphoreType.DMA,
        pltpu.SemaphoreType.DMA,
    ],
)(_gather_body)


def kernel(x, c, we_item):
    del c
    g = _gram(we_item)
    x_flat = x.reshape(TOK)
    return _gather(x_flat, g)
